# baseline (speedup 1.0000x reference)
import functools

import jax
import jax.numpy as jnp
from jax import lax
from jax.experimental import pallas as pl
from jax.experimental.pallas import tpu as pltpu
from jax.experimental.pallas import tpu_sc as plsc

N = 1_000_000
D = 64
B = 98304
TWOB = 2 * B
NLAYER = 5

NC = 2
NS = 16
NW = NC * NS
PER_W = TWOB // NW
CHUNK = 128
NCHUNK = PER_W // CHUNK

BLK = 2048
NBLK = B // BLK


def _sc_gather_fn():
    mesh = plsc.VectorSubcoreMesh(core_axis_name="c", subcore_axis_name="s")

    @functools.partial(
        pl.kernel,
        out_type=jax.ShapeDtypeStruct((TWOB, D), jnp.float32),
        mesh=mesh,
        compiler_params=pltpu.CompilerParams(use_tc_tiling_on_sc=False),
        scratch_types=[
            pltpu.VMEM((NCHUNK, CHUNK), jnp.int32),
            pltpu.VMEM((CHUNK, D), jnp.float32),
            pltpu.VMEM((CHUNK, D), jnp.float32),
            pltpu.VMEM((CHUNK, D), jnp.float32),
            pltpu.VMEM((CHUNK, D), jnp.float32),
            pltpu.SemaphoreType.DMA,
            pltpu.SemaphoreType.DMA,
            pltpu.SemaphoreType.DMA,
            pltpu.SemaphoreType.DMA,
        ],
    )
    def sc_gather(u_hbm, table_hbm, out_hbm, idx_v,
                  rows_a, rows_b, rows_c, rows_d,
                  sem_a, sem_b, sem_c, sem_d):
        wid = lax.axis_index("s") * NC + lax.axis_index("c")
        rowbase = wid * PER_W
        pltpu.sync_copy(u_hbm.at[pl.ds(wid * NCHUNK, NCHUNK)], idx_v)

        bufs = (rows_a, rows_b, rows_c, rows_d)
        sems = (sem_a, sem_b, sem_c, sem_d)

        def gstart(c, b):
            return pltpu.async_copy(table_hbm.at[idx_v.at[c]], bufs[b], sems[b])

        for b in range(4):
            gstart(b, b)

        def step(i, _):
            c0 = 4 * i
            for b in range(4):
                c = c0 + b
                pltpu.make_async_copy(
                    table_hbm.at[idx_v.at[c]], bufs[b], sems[b]).wait()
                pltpu.sync_copy(
                    bufs[b], out_hbm.at[pl.ds(rowbase + c * CHUNK, CHUNK)])

                @pl.when(c + 4 < NCHUNK)
                def _(c=c, b=b):
                    gstart(c + 4, b)

            return 0

        lax.fori_loop(0, NCHUNK // 4, step, 0)

    return sc_gather


def _tc_loss_body(g_ref, lab_ref, lay_ref, w_ref, lt_ref, q_ref, acc_ref):
    blk = g_ref[...]
    ei = blk[:, :D]
    ej = blk[:, D:]
    x = jnp.dot(ei, w_ref[...], preferred_element_type=jnp.float32)
    y = jnp.dot(ej, w_ref[...], preferred_element_type=jnp.float32)
    rxy = jnp.sum(x * y, axis=1, keepdims=True)
    s1 = jnp.dot(x + y, lt_ref[...], preferred_element_type=jnp.float32)
    lay = lay_ref[...]
    onehot = (lay == lax.broadcasted_iota(jnp.int32, (BLK, 8), 1)).astype(jnp.float32)
    inner = rxy + jnp.sum(onehot * (s1 + q_ref[...]), axis=1, keepdims=True)
    t = lab_ref[...] * inner
    part = jnp.sum(jax.nn.log_sigmoid(t))

    @pl.when(pl.program_id(0) == 0)
    def _():
        acc_ref[0, 0] = 0.0

    acc_ref[0, 0] += -part


def kernel(u_i, u_j, this_layer, label, embedding, L_embedding, W):
    m = lax.iota(jnp.int32, TWOB)
    u_all = jnp.where(
        m % 2 == 0,
        jnp.repeat(u_i.astype(jnp.int32), 2),
        jnp.repeat(u_j.astype(jnp.int32), 2),
    ).reshape(TWOB // CHUNK, CHUNK)

    gathered = _sc_gather_fn()(u_all, embedding)
    g2 = gathered.reshape(B, 2 * D)

    lab = label.astype(jnp.float32).reshape(B, 1)
    lay = this_layer.astype(jnp.int32).reshape(B, 1)
    lt = jnp.zeros((D, 8), jnp.float32).at[:, :NLAYER].set(L_embedding.T)
    q = jnp.zeros((1, 8), jnp.float32).at[0, :NLAYER].set(
        jnp.sum(L_embedding * L_embedding, axis=1))

    loss = pl.pallas_call(
        _tc_loss_body,
        grid=(NBLK,),
        in_specs=[
            pl.BlockSpec((BLK, 2 * D), lambda i: (i, 0)),
            pl.BlockSpec((BLK, 1), lambda i: (i, 0)),
            pl.BlockSpec((BLK, 1), lambda i: (i, 0)),
            pl.BlockSpec((D, D), lambda i: (0, 0)),
            pl.BlockSpec((D, 8), lambda i: (0, 0)),
            pl.BlockSpec((1, 8), lambda i: (0, 0)),
        ],
        out_specs=pl.BlockSpec(memory_space=pltpu.SMEM),
        out_shape=jax.ShapeDtypeStruct((1, 1), jnp.float32),
    )(g2, lab, lay, W, lt, q)
    return loss[0, 0]


# speedup vs baseline: 1.0058x; 1.0058x over previous
import functools

import jax
import jax.numpy as jnp
from jax import lax
from jax.experimental import pallas as pl
from jax.experimental.pallas import tpu as pltpu
from jax.experimental.pallas import tpu_sc as plsc

N = 1_000_000
D = 64
B = 98304
TWOB = 2 * B
NLAYER = 5

NC = 2
NS = 16
NW = NC * NS
PER_W = TWOB // NW
CHUNK = 128
NCHUNK = PER_W // CHUNK

BLK = 4096
NBLK = B // BLK


def _sc_gather_fn():
    mesh = plsc.VectorSubcoreMesh(core_axis_name="c", subcore_axis_name="s")

    @functools.partial(
        pl.kernel,
        out_type=jax.ShapeDtypeStruct((TWOB, D), jnp.float32),
        mesh=mesh,
        compiler_params=pltpu.CompilerParams(use_tc_tiling_on_sc=False),
        scratch_types=[
            pltpu.VMEM((NCHUNK, CHUNK), jnp.int32),
            pltpu.VMEM((CHUNK, D), jnp.float32),
            pltpu.VMEM((CHUNK, D), jnp.float32),
            pltpu.VMEM((CHUNK, D), jnp.float32),
            pltpu.VMEM((CHUNK, D), jnp.float32),
            pltpu.SemaphoreType.DMA,
            pltpu.SemaphoreType.DMA,
            pltpu.SemaphoreType.DMA,
            pltpu.SemaphoreType.DMA,
        ],
    )
    def sc_gather(u_hbm, table_hbm, out_hbm, idx_v,
                  rows_a, rows_b, rows_c, rows_d,
                  sem_a, sem_b, sem_c, sem_d):
        wid = lax.axis_index("s") * NC + lax.axis_index("c")
        rowbase = wid * PER_W
        pltpu.sync_copy(u_hbm.at[pl.ds(wid * NCHUNK, NCHUNK)], idx_v)

        bufs = (rows_a, rows_b, rows_c, rows_d)
        sems = (sem_a, sem_b, sem_c, sem_d)

        def gstart(c, b):
            return pltpu.async_copy(table_hbm.at[idx_v.at[c]], bufs[b], sems[b])

        for b in range(4):
            gstart(b, b)

        def step(i, _):
            c0 = 4 * i
            for b in range(4):
                c = c0 + b
                pltpu.make_async_copy(
                    table_hbm.at[idx_v.at[c]], bufs[b], sems[b]).wait()
                pltpu.sync_copy(
                    bufs[b], out_hbm.at[pl.ds(rowbase + c * CHUNK, CHUNK)])

                @pl.when(c + 4 < NCHUNK)
                def _(c=c, b=b):
                    gstart(c + 4, b)

            return 0

        lax.fori_loop(0, NCHUNK // 4, step, 0)

    return sc_gather


def _tc_loss_body(g_ref, lab_ref, lay_ref, w_ref, lt_ref, q_ref, acc_ref):
    blk = g_ref[...]
    ei = blk[:, :D]
    ej = blk[:, D:]
    x = jnp.dot(ei, w_ref[...], preferred_element_type=jnp.float32)
    y = jnp.dot(ej, w_ref[...], preferred_element_type=jnp.float32)
    rxy = jnp.sum(x * y, axis=1, keepdims=True)
    s1 = jnp.dot(x + y, lt_ref[...], preferred_element_type=jnp.float32)
    lay = lay_ref[...]
    onehot = (lay == lax.broadcasted_iota(jnp.int32, (BLK, 8), 1)).astype(jnp.float32)
    inner = rxy + jnp.sum(onehot * (s1 + q_ref[...]), axis=1, keepdims=True)
    t = lab_ref[...] * inner
    part = jnp.sum(jax.nn.log_sigmoid(t))

    @pl.when(pl.program_id(0) == 0)
    def _():
        acc_ref[0, 0] = 0.0

    acc_ref[0, 0] += -part


def kernel(u_i, u_j, this_layer, label, embedding, L_embedding, W):
    m = lax.iota(jnp.int32, TWOB)
    u_all = jnp.where(
        m % 2 == 0,
        jnp.repeat(u_i.astype(jnp.int32), 2),
        jnp.repeat(u_j.astype(jnp.int32), 2),
    ).reshape(TWOB // CHUNK, CHUNK)

    gathered = _sc_gather_fn()(u_all, embedding)
    g2 = gathered.reshape(B, 2 * D)

    lab = label.astype(jnp.float32).reshape(B, 1)
    lay = this_layer.astype(jnp.int32).reshape(B, 1)
    lt = jnp.zeros((D, 8), jnp.float32).at[:, :NLAYER].set(L_embedding.T)
    q = jnp.zeros((1, 8), jnp.float32).at[0, :NLAYER].set(
        jnp.sum(L_embedding * L_embedding, axis=1))

    loss = pl.pallas_call(
        _tc_loss_body,
        grid=(NBLK,),
        in_specs=[
            pl.BlockSpec((BLK, 2 * D), lambda i: (i, 0)),
            pl.BlockSpec((BLK, 1), lambda i: (i, 0)),
            pl.BlockSpec((BLK, 1), lambda i: (i, 0)),
            pl.BlockSpec((D, D), lambda i: (0, 0)),
            pl.BlockSpec((D, 8), lambda i: (0, 0)),
            pl.BlockSpec((1, 8), lambda i: (0, 0)),
        ],
        out_specs=pl.BlockSpec(memory_space=pltpu.SMEM),
        out_shape=jax.ShapeDtypeStruct((1, 1), jnp.float32),
    )(g2, lab, lay, W, lt, q)
    return loss[0, 0]
